# baseline (speedup 1.0000x reference)
import functools

import jax
import jax.numpy as jnp
from jax import lax
from jax.experimental import pallas as pl
from jax.experimental.pallas import tpu as pltpu
from jax.experimental.pallas import tpu_sc as plsc

EMBED = 64
HISTP = 64
HPAD = 56
RING = 8
DEPTH = 6
NC = 2
NS = 16
NW = NC * NS


@functools.lru_cache(maxsize=None)
def _make_gather(bsz: int, hist: int):
    b_per_w = bsz // NW
    n_groups = b_per_w // RING
    assert b_per_w * NW == bsz and n_groups * RING == b_per_w
    assert hist <= HISTP and 2 * hist <= 128
    mesh = plsc.VectorSubcoreMesh(core_axis_name="c", subcore_axis_name="s")

    @functools.partial(
        pl.kernel,
        mesh=mesh,
        compiler_params=pltpu.CompilerParams(use_tc_tiling_on_sc=False),
        out_type=jax.ShapeDtypeStruct((bsz, HISTP, EMBED), jnp.float32),
        scratch_types=(
            [pltpu.VMEM((b_per_w // 2, 128), jnp.int32)]
            + [pltpu.VMEM((HPAD, EMBED), jnp.float32) for _ in range(RING)]
            + [pltpu.SemaphoreType.DMA for _ in range(2 * RING)]
        ),
    )
    def gather(table_hbm, idx_hbm, out_hbm, idx_v, *bufs_and_sems):
        rows = bufs_and_sems[:RING]
        gsem = bufs_and_sems[RING : 2 * RING]
        wsem = bufs_and_sems[2 * RING :]
        wid = lax.axis_index("s") * NC + lax.axis_index("c")
        base = wid * b_per_w
        pltpu.sync_copy(idx_hbm.at[pl.ds(wid * (b_per_w // 2), b_per_w // 2)], idx_v)

        def gather_copy(b, bb):
            idx_list = idx_v.at[bb // 2, pl.ds((bb % 2) * HISTP, HPAD)]
            return pltpu.make_async_copy(
                table_hbm.at[idx_list], rows[b], gsem[b]
            )

        def write_copy(b, bb):
            return pltpu.make_async_copy(
                rows[b], out_hbm.at[base + bb, pl.ds(0, HPAD)], wsem[b]
            )

        for b in range(DEPTH):
            gather_copy(b, b).start()

        def body(g, carry):
            for b in range(RING):
                bb = g * RING + b
                gather_copy(b, bb).wait()
                write_copy(b, bb).start()
                bg = (b + DEPTH) % RING
                nxt = bb + DEPTH

                @pl.when(nxt < b_per_w)
                def _():
                    @pl.when(nxt >= RING)
                    def _():
                        write_copy(bg, nxt - RING).wait()

                    gather_copy(bg, nxt).start()

            return carry

        lax.fori_loop(0, n_groups, body, 0)
        for b in range(RING):
            write_copy(b, b_per_w - RING + b).wait()

    return gather


def kernel(table, type_index):
    bsz, hist = type_index.shape
    idx = type_index.astype(jnp.int32)
    idx = jnp.pad(idx, ((0, 0), (0, HISTP - hist)), constant_values=table.shape[0] - 1)
    idx = idx.reshape(bsz // 2, 2 * HISTP)
    out = _make_gather(bsz, hist)(table, idx)
    return out[:, :hist, :]
